# baseline (speedup 1.0000x reference)
import functools

import jax
import jax.numpy as jnp
from jax import lax
from jax.experimental import pallas as pl
from jax.experimental.pallas import tpu as pltpu
from jax.experimental.pallas import tpu_sc as plsc

N_ROWS = 320000
D_IN = 128
H_DIM = 64
B_SEG = 10000

NC = 2
NS = 16
NW = NC * NS
SEG_PER_W = 320
B_PAD = NW * SEG_PER_W
PTR_W = SEG_PER_W + 32
PTR_PAD = (NW - 1) * SEG_PER_W + PTR_W
LANES = 16
EPS = 1e-5


def _make_seg_mean_sc(feat: int, ch: int, stats_of_means: bool,
                      with_stats: bool = True):
  nreg = feat // LANES

  def body(x_hbm, ptr_hbm, mean_out, *rest):
    if with_stats:
      stats_out, ptr_v, xbuf, seg_v, stats_v, dma_sem = rest
    else:
      ptr_v, xbuf, seg_v, dma_sem = rest
    zero = jnp.zeros((LANES,), jnp.float32)
    nst = nreg if with_stats else 0

    def pv(i):
      return ptr_v[pl.ds(i, LANES)][0]
    cid = lax.axis_index("c")
    sid = lax.axis_index("s")
    wid = sid * NC + cid
    j0 = wid * SEG_PER_W
    pltpu.sync_copy(ptr_hbm.at[pl.ds(j0, PTR_W)], ptr_v)

    r0 = pv(0)
    r1 = pv(SEG_PER_W)
    nch = jnp.maximum((r1 - r0 + ch - 1) // ch, 1)

    def chunk_base(c):
      return jnp.minimum(((r0 + c * ch) // 8) * 8, N_ROWS - (ch + 8))

    def issue(c, bi):
      pltpu.async_copy(x_hbm.at[pl.ds(chunk_base(c), ch + 8)], xbuf.at[bi],
                       dma_sem)

    def drain(bi):
      pltpu.make_async_copy(x_hbm.at[pl.ds(0, ch + 8)], xbuf.at[bi],
                            dma_sem).wait()

    def accum_rows(a, b, base, bi, regs):
      def row_body(r, regs):
        acc, s, q = regs
        off = r - base
        acc, q = list(acc), list(q)
        for k in range(nreg):
          v = xbuf[bi, off, pl.ds(k * LANES, LANES)]
          acc[k] = acc[k] + v
          if with_stats and not stats_of_means:
            q[k] = q[k] + v * v
        return (tuple(acc), s, tuple(q))
      return lax.fori_loop(a, b, row_body, regs)

    issue(0, jnp.int32(0))

    def chunk_body(c, carry):
      j, pos, regs = carry
      lo = r0 + c * ch
      base = chunk_base(c)
      bi = lax.rem(c, 2)
      drain(bi)
      issue(jnp.minimum(c + 1, nch - 1), 1 - bi)
      hi = jnp.minimum(lo + ch, r1)

      cnt = jnp.int32(0)
      lane = lax.iota(jnp.int32, 16)
      for i in range(PTR_W // LANES):
        vec = ptr_v[pl.ds(i * LANES, LANES)]
        m = (vec <= hi) & (lane + (i * LANES) <= SEG_PER_W)
        cnt = cnt + plsc.all_reduce_population_count(m)[0]
      tgt = cnt - 1

      def flush_body(jj, st):
        p, regs = st
        p1 = pv(jj + 1)
        regs = accum_rows(p, p1, base, bi, regs)
        acc, s, q = regs
        seg_n = (p1 - pv(jj)).astype(jnp.float32)
        inv = 1.0 / jnp.maximum(jnp.broadcast_to(seg_n, (LANES,)), 1.0)
        acc = list(acc)
        s = list(s)
        q = list(q)
        if with_stats and stats_of_means:
          live = jnp.where(j0 + jj < B_SEG, 1.0, 0.0)
        for k in range(nreg):
          m = acc[k] * inv
          seg_v[pl.ds(jj * feat + k * LANES, LANES)] = m
          if with_stats and stats_of_means:
            s[k] = s[k] + m * live
            q[k] = q[k] + m * m * live
          elif with_stats:
            s[k] = s[k] + acc[k]
          acc[k] = zero
        return (p1, (tuple(acc), tuple(s), tuple(q)))

      pos, regs = lax.fori_loop(j, tgt, flush_body, (pos, regs))
      regs = accum_rows(pos, hi, base, bi, regs)
      return (tgt, hi, regs)

    init = (jnp.int32(0), r0,
            ((zero,) * nreg, (zero,) * nst, (zero,) * nst))
    j, pos, regs = lax.fori_loop(0, nch, chunk_body, init)
    drain(jnp.int32(0))
    _, s, q = regs
    if with_stats:
      for k in range(nreg):
        stats_v[0, pl.ds(k * LANES, LANES)] = s[k]
        stats_v[1, pl.ds(k * LANES, LANES)] = q[k]
    pltpu.sync_copy(seg_v, mean_out.at[pl.ds(j0 * feat, SEG_PER_W * feat)])
    if with_stats:
      pltpu.sync_copy(stats_v, stats_out.at[wid])

  mesh = plsc.VectorSubcoreMesh(core_axis_name="c", subcore_axis_name="s")
  out_type = [jax.ShapeDtypeStruct((B_PAD * feat,), jnp.float32)]
  scratch = [
      pltpu.VMEM((PTR_W,), jnp.int32),
      pltpu.VMEM((2, ch + 8, feat), jnp.float32),
      pltpu.VMEM((SEG_PER_W * feat,), jnp.float32),
  ]
  if with_stats:
    out_type.append(jax.ShapeDtypeStruct((NW, 2, feat), jnp.float32))
    scratch.append(pltpu.VMEM((2, feat), jnp.float32))
  scratch.append(pltpu.SemaphoreType.DMA)
  return pl.kernel(
      body,
      out_type=out_type,
      mesh=mesh,
      scratch_types=scratch,
      compiler_params=pltpu.CompilerParams(needs_layout_passes=False),
  )


_seg_mean_x = _make_seg_mean_sc(D_IN, 256, stats_of_means=False,
                                with_stats=False)
_seg_mean_h = _make_seg_mean_sc(H_DIM, 256, stats_of_means=True)

_RB = 3200


def _colstats_body(x_ref, out_ref):
  i = pl.program_id(0)
  blk = x_ref[...]
  s = jnp.sum(blk, axis=0, keepdims=True)
  q = jnp.sum(blk * blk, axis=0, keepdims=True)
  t = jnp.concatenate([s, q], axis=0)
  out_ref[...] = jnp.where(i == 0, t, out_ref[...] + t)


def _col_stats(x):
  return pl.pallas_call(
      _colstats_body,
      grid=(N_ROWS // _RB,),
      in_specs=[pl.BlockSpec((_RB, D_IN), lambda i: (i, 0))],
      out_specs=pl.BlockSpec((2, D_IN), lambda i: (0, 0)),
      out_shape=jax.ShapeDtypeStruct((2, D_IN), jnp.float32),
  )(x)


def _stage_b_body(x_ref, w1_ref, b1_ref, g1_ref, be1_ref, st_ref, out_ref):
  s = st_ref[...]
  mu = s[0:1, :] * (1.0 / N_ROWS)
  var = s[1:2, :] * (1.0 / N_ROWS) - mu * mu
  a = g1_ref[...] / jnp.sqrt(var + EPS)
  c = be1_ref[...] - mu * a
  w = w1_ref[...] * a
  beff = lax.dot_general(c, w1_ref[...], (((1,), (1,)), ((), ())),
                         preferred_element_type=jnp.float32) + b1_ref[...]
  h = lax.dot_general(x_ref[...], w, (((1,), (1,)), ((), ())),
                      preferred_element_type=jnp.float32)
  out_ref[...] = jnp.maximum(h + beff, 0.0)


def _stage_d_body(hm_ref, st_ref, g2_ref, be2_ref, w2_ref, b2_ref, out_ref):
  s = jnp.sum(st_ref[...], axis=0)
  mu = s[0:1, :] * (1.0 / B_SEG)
  var = s[1:2, :] * (1.0 / B_SEG) - mu * mu
  a = g2_ref[...] / jnp.sqrt(var + EPS)
  c = be2_ref[...] - mu * a
  hb = hm_ref[...] * a + c
  h2 = lax.dot_general(hb, w2_ref[...], (((1,), (1,)), ((), ())),
                       preferred_element_type=jnp.float32)
  out_ref[...] = jnp.maximum(h2 + b2_ref[...], 0.0)


@jax.jit
def kernel(x, ptr, gamma1, beta1, W1, b1, gamma2, beta2, W2, b2):
  ptr_pad = jnp.concatenate(
      [ptr, jnp.full((PTR_PAD - (B_SEG + 1),), N_ROWS, dtype=ptr.dtype)])

  (mean_x_flat,) = _seg_mean_x(x, ptr_pad)
  xstats = _col_stats(x)

  h = pl.pallas_call(
      _stage_b_body,
      grid=(N_ROWS // _RB,),
      in_specs=[
          pl.BlockSpec((_RB, D_IN), lambda i: (i, 0)),
          pl.BlockSpec((H_DIM, D_IN), lambda i: (0, 0)),
          pl.BlockSpec((1, H_DIM), lambda i: (0, 0)),
          pl.BlockSpec((1, D_IN), lambda i: (0, 0)),
          pl.BlockSpec((1, D_IN), lambda i: (0, 0)),
          pl.BlockSpec((2, D_IN), lambda i: (0, 0)),
      ],
      out_specs=pl.BlockSpec((_RB, H_DIM), lambda i: (i, 0)),
      out_shape=jax.ShapeDtypeStruct((N_ROWS, H_DIM), jnp.float32),
  )(x, W1, b1.reshape(1, H_DIM), gamma1.reshape(1, D_IN),
    beta1.reshape(1, D_IN), xstats)

  hm_flat, hstats = _seg_mean_h(h, ptr_pad)
  hm = hm_flat.reshape(B_PAD, H_DIM)

  h2 = pl.pallas_call(
      _stage_d_body,
      out_shape=jax.ShapeDtypeStruct((B_PAD, H_DIM), jnp.float32),
  )(hm, hstats, gamma2.reshape(1, H_DIM), beta2.reshape(1, H_DIM),
    W2, b2.reshape(1, H_DIM))

  mean_x = mean_x_flat.reshape(B_PAD, D_IN)
  return jnp.concatenate([mean_x[:B_SEG], h2[:B_SEG]], axis=1)


# speedup vs baseline: 1.3474x; 1.3474x over previous
import functools

import jax
import jax.numpy as jnp
from jax import lax
from jax.experimental import pallas as pl
from jax.experimental.pallas import tpu as pltpu
from jax.experimental.pallas import tpu_sc as plsc

N_ROWS = 320000
D_IN = 128
H_DIM = 64
B_SEG = 10000

NC = 2
NS = 16
NW = NC * NS
SEG_PER_W = 320
B_PAD = NW * SEG_PER_W
PTR_W = SEG_PER_W + 32
PTR_PAD = (NW - 1) * SEG_PER_W + PTR_W
LANES = 16
EPS = 1e-5


def _make_seg_mean_sc(feat: int, ch: int, stats_of_means: bool,
                      with_stats: bool = True):
  nreg = feat // LANES

  def body(x_hbm, ptr_hbm, mean_out, *rest):
    if with_stats:
      stats_out, ptr_v, xbuf, seg_v, stats_v, dma_sem = rest
    else:
      ptr_v, xbuf, seg_v, dma_sem = rest
    zero = jnp.zeros((LANES,), jnp.float32)
    nst = nreg if with_stats else 0

    def pv(i):
      return ptr_v[pl.ds(i, LANES)][0]
    cid = lax.axis_index("c")
    sid = lax.axis_index("s")
    wid = sid * NC + cid
    j0 = wid * SEG_PER_W
    pltpu.sync_copy(ptr_hbm.at[pl.ds(j0, PTR_W)], ptr_v)

    r0 = pv(0)
    r1 = pv(SEG_PER_W)
    nch = jnp.maximum((r1 - r0 + ch - 1) // ch, 1)

    def chunk_base(c):
      return jnp.minimum(((r0 + c * ch) // 8) * 8, N_ROWS - (ch + 8))

    def issue(c, bi):
      pltpu.async_copy(x_hbm.at[pl.ds(chunk_base(c), ch + 8)], xbuf.at[bi],
                       dma_sem)

    def drain(bi):
      pltpu.make_async_copy(x_hbm.at[pl.ds(0, ch + 8)], xbuf.at[bi],
                            dma_sem).wait()

    def accum_rows(a, b, base, bi, regs):
      npair = (b - a + 1) // 2

      def pair_body(i, regs):
        acc, s, q = regs
        ra = a + 2 * i
        rb = jnp.minimum(ra + 1, b - 1)
        w = jnp.broadcast_to(
            jnp.where(ra + 1 < b, 1.0, 0.0).astype(jnp.float32), (LANES,))
        oa = ra - base
        ob = rb - base
        acc, q = list(acc), list(q)
        for k in range(nreg):
          v0 = xbuf[bi, oa, pl.ds(k * LANES, LANES)]
          v1 = xbuf[bi, ob, pl.ds(k * LANES, LANES)]
          v1w = v1 * w
          acc[k] = acc[k] + (v0 + v1w)
          if with_stats and not stats_of_means:
            q[k] = q[k] + (v0 * v0 + v1w * v1w)
        return (tuple(acc), s, tuple(q))
      return lax.fori_loop(0, npair, pair_body, regs)

    issue(0, jnp.int32(0))

    def chunk_body(c, carry):
      j, pos, regs = carry
      lo = r0 + c * ch
      base = chunk_base(c)
      bi = lax.rem(c, 2)
      drain(bi)
      issue(jnp.minimum(c + 1, nch - 1), 1 - bi)
      hi = jnp.minimum(lo + ch, r1)

      cnt = jnp.int32(0)
      lane = lax.iota(jnp.int32, 16)
      for i in range(PTR_W // LANES):
        vec = ptr_v[pl.ds(i * LANES, LANES)]
        m = (vec <= hi) & (lane + (i * LANES) <= SEG_PER_W)
        cnt = cnt + plsc.all_reduce_population_count(m)[0]
      tgt = cnt - 1

      def flush_body(jj, st):
        p, regs = st
        p1 = pv(jj + 1)
        regs = accum_rows(p, p1, base, bi, regs)
        acc, s, q = regs
        seg_n = (p1 - pv(jj)).astype(jnp.float32)
        inv = 1.0 / jnp.maximum(jnp.broadcast_to(seg_n, (LANES,)), 1.0)
        acc = list(acc)
        s = list(s)
        q = list(q)
        if with_stats and stats_of_means:
          live = jnp.where(j0 + jj < B_SEG, 1.0, 0.0)
        for k in range(nreg):
          m = acc[k] * inv
          seg_v[jj, pl.ds(k * LANES, LANES)] = m
          if with_stats and stats_of_means:
            s[k] = s[k] + m * live
            q[k] = q[k] + m * m * live
          elif with_stats:
            s[k] = s[k] + acc[k]
          acc[k] = zero
        return (p1, (tuple(acc), tuple(s), tuple(q)))

      pos, regs = lax.fori_loop(j, tgt, flush_body, (pos, regs))
      regs = accum_rows(pos, hi, base, bi, regs)
      return (tgt, hi, regs)

    init = (jnp.int32(0), r0,
            ((zero,) * nreg, (zero,) * nst, (zero,) * nst))
    j, pos, regs = lax.fori_loop(0, nch, chunk_body, init)
    drain(jnp.int32(0))
    _, s, q = regs
    if with_stats:
      for k in range(nreg):
        stats_v[0, pl.ds(k * LANES, LANES)] = s[k]
        stats_v[1, pl.ds(k * LANES, LANES)] = q[k]
    pltpu.sync_copy(seg_v, mean_out.at[pl.ds(j0, SEG_PER_W)])
    if with_stats:
      pltpu.sync_copy(stats_v, stats_out.at[wid])

  mesh = plsc.VectorSubcoreMesh(core_axis_name="c", subcore_axis_name="s")
  out_type = [jax.ShapeDtypeStruct((B_PAD, feat), jnp.float32)]
  scratch = [
      pltpu.VMEM((PTR_W,), jnp.int32),
      pltpu.VMEM((2, ch + 8, feat), jnp.float32),
      pltpu.VMEM((SEG_PER_W, feat), jnp.float32),
  ]
  if with_stats:
    out_type.append(jax.ShapeDtypeStruct((NW, 2, feat), jnp.float32))
    scratch.append(pltpu.VMEM((2, feat), jnp.float32))
  scratch.append(pltpu.SemaphoreType.DMA)
  return pl.kernel(
      body,
      out_type=out_type,
      mesh=mesh,
      scratch_types=scratch,
      compiler_params=pltpu.CompilerParams(needs_layout_passes=False),
  )


_seg_mean_x = _make_seg_mean_sc(D_IN, 256, stats_of_means=False)
_seg_mean_h = _make_seg_mean_sc(H_DIM, 256, stats_of_means=True)

_RB = 3200


def _stage_b_body(x_ref, w1_ref, b1_ref, g1_ref, be1_ref, st_ref, out_ref):
  s = jnp.sum(st_ref[...], axis=0)
  mu = s[0:1, :] * (1.0 / N_ROWS)
  var = s[1:2, :] * (1.0 / N_ROWS) - mu * mu
  a = g1_ref[...] / jnp.sqrt(var + EPS)
  c = be1_ref[...] - mu * a
  w = w1_ref[...] * a
  beff = lax.dot_general(c, w1_ref[...], (((1,), (1,)), ((), ())),
                         preferred_element_type=jnp.float32) + b1_ref[...]
  h = lax.dot_general(x_ref[...], w, (((1,), (1,)), ((), ())),
                      preferred_element_type=jnp.float32)
  out_ref[...] = jnp.maximum(h + beff, 0.0)


def _stage_d_body(mx_ref, hm_ref, st_ref, g2_ref, be2_ref, w2_ref, b2_ref,
                  out_ref):
  s = jnp.sum(st_ref[...], axis=0)
  mu = s[0:1, :] * (1.0 / B_SEG)
  var = s[1:2, :] * (1.0 / B_SEG) - mu * mu
  a = g2_ref[...] / jnp.sqrt(var + EPS)
  c = be2_ref[...] - mu * a
  hb = hm_ref[...] * a + c
  h2 = lax.dot_general(hb, w2_ref[...], (((1,), (1,)), ((), ())),
                       preferred_element_type=jnp.float32)
  h2 = jnp.maximum(h2 + b2_ref[...], 0.0)
  out_ref[...] = jnp.concatenate(
      [mx_ref[...][:B_SEG], h2[:B_SEG]], axis=1)


@jax.jit
def kernel(x, ptr, gamma1, beta1, W1, b1, gamma2, beta2, W2, b2):
  ptr_pad = jnp.concatenate(
      [ptr, jnp.full((PTR_PAD - (B_SEG + 1),), N_ROWS, dtype=ptr.dtype)])

  mean_x, xstats = _seg_mean_x(x, ptr_pad)

  h = pl.pallas_call(
      _stage_b_body,
      grid=(N_ROWS // _RB,),
      in_specs=[
          pl.BlockSpec((_RB, D_IN), lambda i: (i, 0)),
          pl.BlockSpec((H_DIM, D_IN), lambda i: (0, 0)),
          pl.BlockSpec((1, H_DIM), lambda i: (0, 0)),
          pl.BlockSpec((1, D_IN), lambda i: (0, 0)),
          pl.BlockSpec((1, D_IN), lambda i: (0, 0)),
          pl.BlockSpec((NW, 2, D_IN), lambda i: (0, 0, 0)),
      ],
      out_specs=pl.BlockSpec((_RB, H_DIM), lambda i: (i, 0)),
      out_shape=jax.ShapeDtypeStruct((N_ROWS, H_DIM), jnp.float32),
  )(x, W1, b1.reshape(1, H_DIM), gamma1.reshape(1, D_IN),
    beta1.reshape(1, D_IN), xstats)

  hm, hstats = _seg_mean_h(h, ptr_pad)

  return pl.pallas_call(
      _stage_d_body,
      out_shape=jax.ShapeDtypeStruct((B_SEG, D_IN + H_DIM), jnp.float32),
  )(mean_x, hm, hstats, gamma2.reshape(1, H_DIM), beta2.reshape(1, H_DIM),
    W2, b2.reshape(1, H_DIM))
